# Initial kernel scaffold; baseline (speedup 1.0000x reference)
import jax
import jax.numpy as jnp
from jax.experimental import pallas as pl


def kernel(x, edge_index, edge_weight, idx, W1, b1, W2, b2):
    raise NotImplementedError("write your pallas kernel here")


# speedup vs baseline: 2.6277x; 2.6277x over previous
import functools

import jax
import jax.numpy as jnp
from jax import lax
from jax.experimental import pallas as pl
from jax.experimental.pallas import tpu as pltpu
from jax.experimental.pallas import tpu_sc as plsc

N_NODES = 10000
N_EDGES = 320000
D = 128
IDX_N = 2048

NC = 2
NS = 16
NW = NC * NS

CHUNK = 128
E_PAD = 327680
E_PER_CORE = E_PAD // NC
E_PER_TILE = E_PER_CORE // NS
NCHUNK = E_PER_TILE // CHUNK

N_ACC = 10240
ROWS_PER_TILE = N_ACC // NS
ZROWS = 128

_mesh = plsc.VectorSubcoreMesh(
    core_axis_name="c", subcore_axis_name="s", num_cores=NC, num_subcores=NS
)


def _linear_body(x_ref, wt_ref, b_ref, o_ref):
    o_ref[...] = (
        jnp.dot(x_ref[...], wt_ref[...], preferred_element_type=jnp.float32)
        + b_ref[...]
    )


def _linear(x, wt, b):
    m = x.shape[0]
    bm = 1000
    return pl.pallas_call(
        _linear_body,
        grid=(m // bm,),
        in_specs=[
            pl.BlockSpec((bm, D), lambda i: (i, 0)),
            pl.BlockSpec((D, D), lambda i: (0, 0)),
            pl.BlockSpec((1, D), lambda i: (0, 0)),
        ],
        out_specs=pl.BlockSpec((bm, D), lambda i: (i, 0)),
        out_shape=jax.ShapeDtypeStruct((m, D), jnp.float32),
    )(x, wt, b)


def _fused_body(a_ref, wt_ref, b_ref, o_ref):
    h = jnp.maximum(a_ref[0] + a_ref[1], 0.0)
    o_ref[...] = (
        jnp.dot(h, wt_ref[...], preferred_element_type=jnp.float32) + b_ref[...]
    )


def _relu_add_linear(a, wt, b):
    m = a.shape[1]
    bm = m // 10
    return pl.pallas_call(
        _fused_body,
        grid=(m // bm,),
        in_specs=[
            pl.BlockSpec((2, bm, D), lambda i: (0, i, 0)),
            pl.BlockSpec((D, D), lambda i: (0, 0)),
            pl.BlockSpec((1, D), lambda i: (0, 0)),
        ],
        out_specs=pl.BlockSpec((bm, D), lambda i: (i, 0)),
        out_shape=jax.ShapeDtypeStruct((m, D), jnp.float32),
    )(a, wt, b)


@functools.partial(
    pl.kernel,
    out_type=jax.ShapeDtypeStruct((NC, N_ACC, D), jnp.float32),
    mesh=_mesh,
    scratch_types=[
        pltpu.VMEM((CHUNK,), jnp.int32),
        pltpu.VMEM((CHUNK,), jnp.int32),
        pltpu.VMEM((CHUNK,), jnp.float32),
        pltpu.VMEM((CHUNK, D), jnp.float32),
        pltpu.VMEM((ZROWS, D), jnp.float32),
        pltpu.VMEM_SHARED((N_ACC, D), jnp.float32),
        pltpu.SemaphoreType.DMA,
    ],
)
def _spmm_sc(h_hbm, src_hbm, dst_hbm, w_hbm, out_hbm,
             src_v, dst_v, w_v, rows_v, zero_v, acc_sh, sem):
    c = lax.axis_index("c")
    s = lax.axis_index("s")

    def zbody(i, _):
        zero_v[i // 8, pl.ds((i % 8) * 16, 16)] = jnp.zeros((16,), jnp.float32)
        return 0
    lax.fori_loop(0, ZROWS * 8, zbody, 0)
    for k in range(ROWS_PER_TILE // ZROWS):
        pltpu.sync_copy(
            zero_v, acc_sh.at[pl.ds(s * ROWS_PER_TILE + k * ZROWS, ZROWS)]
        )
    plsc.subcore_barrier()

    base = c * E_PER_CORE + s * E_PER_TILE

    def chunk_body(j, _):
        off = pl.multiple_of(base + j * CHUNK, CHUNK)
        pltpu.sync_copy(src_hbm.at[pl.ds(off, CHUNK)], src_v)
        pltpu.sync_copy(dst_hbm.at[pl.ds(off, CHUNK)], dst_v)
        pltpu.sync_copy(w_hbm.at[pl.ds(off, CHUNK)], w_v)
        pltpu.async_copy(h_hbm.at[src_v], rows_v, sem).wait()

        def group_body(g, _):
            wv = w_v[pl.ds(g * 16, 16)]
            for t in range(16):
                w = wv[t]
                row = g * 16 + t
                for f in range(8):
                    sl = pl.ds(f * 16, 16)
                    rows_v[row, sl] = rows_v[row, sl] * w
            return 0
        lax.fori_loop(0, CHUNK // 16, group_body, 0)

        pltpu.sync_copy(rows_v, acc_sh.at[dst_v], add=True)
        return 0
    lax.fori_loop(0, NCHUNK, chunk_body, 0)

    plsc.subcore_barrier()
    pltpu.sync_copy(
        acc_sh.at[pl.ds(s * ROWS_PER_TILE, ROWS_PER_TILE)],
        out_hbm.at[c, pl.ds(s * ROWS_PER_TILE, ROWS_PER_TILE)],
    )


B_PER_W = IDX_N // NW


@functools.partial(
    pl.kernel,
    out_type=jax.ShapeDtypeStruct((IDX_N, D), jnp.float32),
    mesh=_mesh,
    scratch_types=[
        pltpu.VMEM((B_PER_W,), jnp.int32),
        pltpu.VMEM((B_PER_W, D), jnp.float32),
        pltpu.VMEM((B_PER_W, D), jnp.float32),
        pltpu.SemaphoreType.DMA,
    ],
)
def _gather_add(g0_hbm, g1_hbm, idx_hbm, out_hbm, idx_v, r0, r1, sem):
    c = lax.axis_index("c")
    s = lax.axis_index("s")
    wid = c * NS + s
    base = pl.multiple_of(wid * B_PER_W, B_PER_W)
    pltpu.sync_copy(idx_hbm.at[pl.ds(base, B_PER_W)], idx_v)
    pltpu.async_copy(g0_hbm.at[idx_v], r0, sem).wait()
    pltpu.async_copy(g1_hbm.at[idx_v], r1, sem).wait()

    def add_body(e, _):
        for f in range(8):
            sl = pl.ds(f * 16, 16)
            r0[e, sl] = r0[e, sl] + r1[e, sl]
        return 0
    lax.fori_loop(0, B_PER_W, add_body, 0)
    pltpu.sync_copy(r0, out_hbm.at[pl.ds(base, B_PER_W)])


def kernel(x, edge_index, edge_weight, idx, W1, b1, W2, b2):
    pad = E_PAD - N_EDGES
    src = jnp.pad(edge_index[1], (0, pad))
    dst = jnp.pad(edge_index[0], (0, pad))
    w = jnp.pad(edge_weight, (0, pad))

    h1 = _linear(x, W1.T, b1.reshape(1, D))
    a1 = _spmm_sc(h1, src, dst, w)
    h2 = _relu_add_linear(a1, W2.T, b2.reshape(1, D))
    a2 = _spmm_sc(h2, src, dst, w)
    return _gather_add(a2[0], a2[1], idx)
